# baseline (speedup 1.0000x reference)
import functools

import jax
import jax.numpy as jnp
from jax import lax
from jax.experimental import pallas as pl
from jax.experimental.pallas import tpu as pltpu
from jax.experimental.pallas import tpu_sc as plsc

B, S, D = 4, 4096, 768
LANES = 16
NW = 32
ROWS_PER_W = (B * S) // NW
CHUNK = 64
NCHUNK = ROWS_PER_W // CHUNK
VPR = D // LANES

_mesh = plsc.VectorSubcoreMesh(core_axis_name="c", subcore_axis_name="s")


@functools.partial(
    pl.kernel,
    mesh=_mesh,
    out_type=jax.ShapeDtypeStruct((B * S, D), jnp.float32),
    scratch_types=[
        pltpu.VMEM((ROWS_PER_W,), jnp.int32),
        pltpu.VMEM((CHUNK, D), jnp.float32),
        pltpu.VMEM((CHUNK, D), jnp.float32),
        pltpu.SemaphoreType.DMA,
    ],
)
def _embed(x_hbm, table_hbm, pos_hbm, out_hbm, idx_v, rows_v, pos_v, sem):
    cid = lax.axis_index("c")
    sid = lax.axis_index("s")
    wid = sid * 2 + cid
    base = wid * ROWS_PER_W
    pltpu.sync_copy(x_hbm.at[pl.ds(base, ROWS_PER_W)], idx_v)
    seq_base = lax.rem(base, S)

    def chunk_body(g, carry):
        rbase = g * CHUNK
        gcp = pltpu.async_copy(table_hbm.at[idx_v.at[pl.ds(rbase, CHUNK)]], rows_v, sem)
        pltpu.sync_copy(pos_hbm.at[pl.ds(seq_base + rbase, CHUNK)], pos_v)
        gcp.wait()

        def add_row(j, c2):
            def add_vec(k, c3):
                off = k * LANES
                rows_v[j, pl.ds(off, LANES)] = (
                    rows_v[j, pl.ds(off, LANES)] + pos_v[j, pl.ds(off, LANES)]
                )
                return c3

            return lax.fori_loop(0, VPR, add_vec, c2)

        lax.fori_loop(0, CHUNK, add_row, 0)
        pltpu.sync_copy(rows_v, out_hbm.at[pl.ds(base + rbase, CHUNK)])
        return carry

    lax.fori_loop(0, NCHUNK, chunk_body, 0)


def kernel(x, table, pos_encoding):
    xf = x.reshape(-1).astype(jnp.int32)
    out = _embed(xf, table, pos_encoding)
    return out.reshape(B, S, D)


# speedup vs baseline: 2.8531x; 2.8531x over previous
import functools

import jax
import jax.numpy as jnp
from jax import lax
from jax.experimental import pallas as pl
from jax.experimental.pallas import tpu as pltpu
from jax.experimental.pallas import tpu_sc as plsc

B, S, D = 4, 4096, 768
LANES = 16
VPR = D // LANES
NW = 32
SEQ_PER_W = S // NW
CS = 8
NCHUNK = SEQ_PER_W // CS
NBUF = 4
AHEAD = 2

_mesh = plsc.VectorSubcoreMesh(core_axis_name="c", subcore_axis_name="s")

_scratch = (
    [pltpu.VMEM((B * SEQ_PER_W,), jnp.int32)]
    + [pltpu.VMEM((B * CS, D), jnp.float32) for _ in range(NBUF)]
    + [pltpu.VMEM((CS, D), jnp.float32) for _ in range(NBUF)]
    + [pltpu.SemaphoreType.DMA for _ in range(3 * NBUF)]
)


@functools.partial(
    pl.kernel,
    mesh=_mesh,
    out_type=jax.ShapeDtypeStruct((B * S, D), jnp.float32),
    scratch_types=_scratch,
)
def _embed(x_hbm, table_hbm, pos_hbm, out_hbm, idx_v, *bufs):
    rows_bufs = bufs[:NBUF]
    pos_bufs = bufs[NBUF:2 * NBUF]
    gsem = bufs[2 * NBUF:3 * NBUF]
    psem = bufs[3 * NBUF:4 * NBUF]
    ssem = bufs[4 * NBUF:5 * NBUF]

    cid = lax.axis_index("c")
    sid = lax.axis_index("s")
    wid = sid * 2 + cid
    wseq = wid * SEQ_PER_W

    for b in range(B):
        pltpu.sync_copy(
            x_hbm.at[pl.ds(b * S + wseq, SEQ_PER_W)],
            idx_v.at[pl.ds(b * SEQ_PER_W, SEQ_PER_W)],
        )

    gather_cps = [None] * NBUF
    pos_cps = [None] * NBUF
    store_cps = [None] * NBUF

    def fire(g):
        buf = g % NBUF
        if store_cps[buf] is not None:
            for c in store_cps[buf]:
                c.wait()
        pos_cps[buf] = pltpu.async_copy(
            pos_hbm.at[pl.ds(wseq + g * CS, CS)], pos_bufs[buf], psem[buf]
        )
        gather_cps[buf] = [
            pltpu.async_copy(
                table_hbm.at[idx_v.at[pl.ds(b * SEQ_PER_W + g * CS, CS)]],
                rows_bufs[buf].at[pl.ds(b * CS, CS)],
                gsem[buf],
            )
            for b in range(B)
        ]

    for g in range(min(AHEAD, NCHUNK)):
        fire(g)

    for g in range(NCHUNK):
        buf = g % NBUF
        for c in gather_cps[buf]:
            c.wait()
        pos_cps[buf].wait()
        if g + AHEAD < NCHUNK:
            fire(g + AHEAD)

        rows = rows_bufs[buf]
        pos = pos_bufs[buf]

        def add_row(j, c2, rows=rows, pos=pos):
            def add_vec(k, c3):
                off = k * LANES
                p = pos[j, pl.ds(off, LANES)]
                for b in range(B):
                    plsc.addupdate(rows.at[b * CS + j, pl.ds(off, LANES)], p)
                return c3

            return lax.fori_loop(0, VPR, add_vec, c2)

        lax.fori_loop(0, CS, add_row, 0)

        store_cps[buf] = [
            pltpu.async_copy(
                rows.at[pl.ds(b * CS, CS)],
                out_hbm.at[pl.ds(b * S + wseq + g * CS, CS)],
                ssem[buf],
            )
            for b in range(B)
        ]

    for cps in store_cps:
        if cps is not None:
            for c in cps:
                c.wait()


def kernel(x, table, pos_encoding):
    xf = x.reshape(-1).astype(jnp.int32)
    out = _embed(xf, table, pos_encoding)
    return out.reshape(B, S, D)


# speedup vs baseline: 2.8757x; 1.0079x over previous
import functools

import jax
import jax.numpy as jnp
from jax import lax
from jax.experimental import pallas as pl
from jax.experimental.pallas import tpu as pltpu
from jax.experimental.pallas import tpu_sc as plsc

B, S, D = 4, 4096, 768
LANES = 16
VPR = D // LANES
NW = 32
SEQ_PER_W = S // NW
CS = 8
NCHUNK = SEQ_PER_W // CS
NBUF = 4
AHEAD = 2

_mesh = plsc.VectorSubcoreMesh(core_axis_name="c", subcore_axis_name="s")

_scratch = (
    [pltpu.VMEM((B * SEQ_PER_W,), jnp.int32)]
    + [pltpu.VMEM((B * CS, D), jnp.float32) for _ in range(NBUF)]
    + [pltpu.VMEM((CS, D), jnp.float32) for _ in range(NBUF)]
    + [pltpu.SemaphoreType.DMA for _ in range(3 * NBUF)]
)


@functools.partial(
    pl.kernel,
    mesh=_mesh,
    out_type=jax.ShapeDtypeStruct((B * S, D), jnp.float32),
    scratch_types=_scratch,
)
def _embed(x_hbm, table_hbm, pos_hbm, out_hbm, idx_v, *bufs):
    rows_bufs = bufs[:NBUF]
    pos_bufs = bufs[NBUF:2 * NBUF]
    gsem = bufs[2 * NBUF:3 * NBUF]
    psem = bufs[3 * NBUF:4 * NBUF]
    ssem = bufs[4 * NBUF:5 * NBUF]

    cid = lax.axis_index("c")
    sid = lax.axis_index("s")
    wid = sid * 2 + cid
    wseq = wid * SEQ_PER_W

    idx_cps = []
    for g in range(NCHUNK):
        for b in range(B):
            idx_cps.append(
                pltpu.async_copy(
                    x_hbm.at[pl.ds(b * S + wseq + g * CS, CS)],
                    idx_v.at[pl.ds(g * B * CS + b * CS, CS)],
                    gsem[0],
                )
            )
    for c in idx_cps:
        c.wait()

    gather_cps = [None] * NBUF
    pos_cps = [None] * NBUF
    store_cps = [None] * NBUF

    def fire(g):
        buf = g % NBUF
        if store_cps[buf] is not None:
            for c in store_cps[buf]:
                c.wait()
        pos_cps[buf] = pltpu.async_copy(
            pos_hbm.at[pl.ds(wseq + g * CS, CS)], pos_bufs[buf], psem[buf]
        )
        gather_cps[buf] = [
            pltpu.async_copy(
                table_hbm.at[idx_v.at[pl.ds(g * B * CS, B * CS)]],
                rows_bufs[buf],
                gsem[buf],
            )
        ]

    for g in range(min(AHEAD, NCHUNK)):
        fire(g)

    for g in range(NCHUNK):
        buf = g % NBUF
        for c in gather_cps[buf]:
            c.wait()
        pos_cps[buf].wait()
        if g + AHEAD < NCHUNK:
            fire(g + AHEAD)

        rows = rows_bufs[buf]
        pos = pos_bufs[buf]

        def add_row(j, c2, rows=rows, pos=pos):
            def add_vec(k, c3):
                off = k * LANES
                p = pos[j, pl.ds(off, LANES)]
                for b in range(B):
                    plsc.addupdate(rows.at[b * CS + j, pl.ds(off, LANES)], p)
                return c3

            return lax.fori_loop(0, VPR, add_vec, c2)

        lax.fori_loop(0, CS, add_row, 0)

        store_cps[buf] = [
            pltpu.async_copy(
                rows.at[pl.ds(b * CS, CS)],
                out_hbm.at[pl.ds(b * S + wseq + g * CS, CS)],
                ssem[buf],
            )
            for b in range(B)
        ]

    for cps in store_cps:
        if cps is not None:
            for c in cps:
                c.wait()


def kernel(x, table, pos_encoding):
    xf = x.reshape(-1).astype(jnp.int32)
    out = _embed(xf, table, pos_encoding)
    return out.reshape(B, S, D)


# speedup vs baseline: 3.0251x; 1.0520x over previous
import functools

import jax
import jax.numpy as jnp
from jax import lax
from jax.experimental import pallas as pl
from jax.experimental.pallas import tpu as pltpu
from jax.experimental.pallas import tpu_sc as plsc

B, S, D = 4, 4096, 768
LANES = 16
VPR = D // LANES
NW = 32
SEQ_PER_W = S // NW
CS = 8
NCHUNK = SEQ_PER_W // CS
NBUF = 4
AHEAD = 2

_mesh = plsc.VectorSubcoreMesh(core_axis_name="c", subcore_axis_name="s")

_scratch = (
    [pltpu.VMEM((NCHUNK * B * CS,), jnp.int32)]
    + [pltpu.VMEM((B * CS, D), jnp.float32) for _ in range(NBUF)]
    + [pltpu.VMEM((CS, D), jnp.float32) for _ in range(NBUF)]
    + [pltpu.SemaphoreType.DMA for _ in range(3 * NBUF)]
)


@functools.partial(
    pl.kernel,
    mesh=_mesh,
    out_type=jax.ShapeDtypeStruct((B * S, D), jnp.float32),
    scratch_types=_scratch,
)
def _embed(x_hbm, table_hbm, pos_hbm, out_hbm, idx_v, *bufs):
    rows_bufs = bufs[:NBUF]
    pos_bufs = bufs[NBUF:2 * NBUF]
    gsem = bufs[2 * NBUF:3 * NBUF]
    psem = bufs[3 * NBUF:4 * NBUF]
    ssem = bufs[4 * NBUF:5 * NBUF]

    cid = lax.axis_index("c")
    sid = lax.axis_index("s")
    wid = sid * 2 + cid
    wseq = wid * SEQ_PER_W

    @pl.loop(0, NCHUNK)
    def _stage(g):
        for b in range(B):
            pltpu.async_copy(
                x_hbm.at[pl.ds(b * S + wseq + g * CS, CS)],
                idx_v.at[pl.ds(g * B * CS + b * CS, CS)],
                gsem[0],
            )

    @pl.loop(0, NCHUNK * B)
    def _stage_drain(t):
        pltpu.make_async_copy(
            x_hbm.at[pl.ds(0, CS)], idx_v.at[pl.ds(0, CS)], gsem[0]
        ).wait()

    def fire(g, j):
        pltpu.async_copy(
            pos_hbm.at[pl.ds(wseq + g * CS, CS)], pos_bufs[j], psem[j]
        )
        pltpu.async_copy(
            table_hbm.at[idx_v.at[pl.ds(g * B * CS, B * CS)]],
            rows_bufs[j],
            gsem[j],
        )

    for g in range(AHEAD):
        fire(g, g % NBUF)

    @pl.loop(0, NCHUNK, step=NBUF)
    def _main(gout):
        for i in range(NBUF):
            g = gout + i
            pltpu.make_async_copy(
                out_hbm.at[pl.ds(0, B * CS)], rows_bufs[i], gsem[i]
            ).wait()
            pltpu.make_async_copy(
                out_hbm.at[pl.ds(0, CS)], pos_bufs[i], psem[i]
            ).wait()

            j = (i + AHEAD) % NBUF

            @pl.when(g + AHEAD < NCHUNK)
            def _fire_ahead(g=g, j=j):
                @pl.when(g + AHEAD >= NBUF)
                def _drain_store():
                    for b in range(B):
                        pltpu.make_async_copy(
                            rows_bufs[j].at[pl.ds(b * CS, CS)],
                            out_hbm.at[pl.ds(b * S, CS)],
                            ssem[j],
                        ).wait()

                fire(g + AHEAD, j)

            rows = rows_bufs[i]
            pos = pos_bufs[i]

            def add_row(jj, c2, rows=rows, pos=pos):
                def add_vec(k, c3):
                    off = k * LANES
                    p = pos[jj, pl.ds(off, LANES)]
                    for b in range(B):
                        plsc.addupdate(rows.at[b * CS + jj, pl.ds(off, LANES)], p)
                    return c3

                return lax.fori_loop(0, VPR, add_vec, c2)

            lax.fori_loop(0, CS, add_row, 0)

            for b in range(B):
                pltpu.async_copy(
                    rows.at[pl.ds(b * CS, CS)],
                    out_hbm.at[pl.ds(b * S + wseq + g * CS, CS)],
                    ssem[i],
                )

    for i in range(NBUF):
        for b in range(B):
            pltpu.make_async_copy(
                rows_bufs[i].at[pl.ds(b * CS, CS)],
                out_hbm.at[pl.ds(b * S, CS)],
                ssem[i],
            ).wait()


def kernel(x, table, pos_encoding):
    xf = x.reshape(-1).astype(jnp.int32)
    out = _embed(xf, table, pos_encoding)
    return out.reshape(B, S, D)


# speedup vs baseline: 3.0397x; 1.0048x over previous
import functools

import jax
import jax.numpy as jnp
from jax import lax
from jax.experimental import pallas as pl
from jax.experimental.pallas import tpu as pltpu
from jax.experimental.pallas import tpu_sc as plsc

B, S, D = 4, 4096, 768
LANES = 16
VPR = D // LANES
NW = 32
SEQ_PER_W = S // NW
CS = 8
NCHUNK = SEQ_PER_W // CS
NBUF = 4
AHEAD = 2

_mesh = plsc.VectorSubcoreMesh(core_axis_name="c", subcore_axis_name="s")

_scratch = (
    [pltpu.VMEM((NCHUNK * B * CS,), jnp.int32)]
    + [pltpu.VMEM((B * CS, D), jnp.float32) for _ in range(NBUF)]
    + [pltpu.VMEM((CS, D), jnp.float32) for _ in range(NBUF)]
    + [pltpu.SemaphoreType.DMA for _ in range(3 * NBUF)]
)


@functools.partial(
    pl.kernel,
    mesh=_mesh,
    out_type=jax.ShapeDtypeStruct((B, S, D), jnp.float32),
    scratch_types=_scratch,
)
def _embed(x_hbm, table_hbm, pos_hbm, out_hbm, idx_v, *bufs):
    rows_bufs = bufs[:NBUF]
    pos_bufs = bufs[NBUF:2 * NBUF]
    gsem = bufs[2 * NBUF:3 * NBUF]
    psem = bufs[3 * NBUF:4 * NBUF]
    ssem = bufs[4 * NBUF:5 * NBUF]

    cid = lax.axis_index("c")
    sid = lax.axis_index("s")
    wid = sid * 2 + cid
    wseq = wid * SEQ_PER_W

    def stage(g):
        for b in range(B):
            pltpu.async_copy(
                x_hbm.at[b, pl.ds(wseq + g * CS, CS)],
                idx_v.at[pl.ds(g * B * CS + b * CS, CS)],
                gsem[0],
            )

    def stage_drain(n):
        @pl.loop(0, n * B)
        def _(t):
            pltpu.make_async_copy(
                x_hbm.at[0, pl.ds(0, CS)], idx_v.at[pl.ds(0, CS)], gsem[0]
            ).wait()

    def fire(g, j):
        pltpu.async_copy(
            pos_hbm.at[pl.ds(wseq + g * CS, CS)], pos_bufs[j], psem[j]
        )
        pltpu.async_copy(
            table_hbm.at[idx_v.at[pl.ds(g * B * CS, B * CS)]],
            rows_bufs[j],
            gsem[j],
        )

    for g in range(AHEAD):
        stage(g)
    stage_drain(AHEAD)
    for g in range(AHEAD):
        fire(g, g % NBUF)

    @pl.loop(AHEAD, NCHUNK)
    def _stage_rest(g):
        stage(g)

    stage_drain(NCHUNK - AHEAD)

    @pl.loop(0, NCHUNK, step=NBUF)
    def _main(gout):
        for i in range(NBUF):
            g = gout + i
            pltpu.make_async_copy(
                table_hbm.at[pl.ds(0, B * CS)], rows_bufs[i], gsem[i]
            ).wait()
            pltpu.make_async_copy(
                pos_hbm.at[pl.ds(0, CS)], pos_bufs[i], psem[i]
            ).wait()

            j = (i + AHEAD) % NBUF

            @pl.when(g + AHEAD < NCHUNK)
            def _fire_ahead(g=g, j=j):
                @pl.when(g + AHEAD >= NBUF)
                def _drain_store():
                    for b in range(B):
                        pltpu.make_async_copy(
                            rows_bufs[j].at[pl.ds(b * CS, CS)],
                            out_hbm.at[b, pl.ds(0, CS)],
                            ssem[j],
                        ).wait()

                fire(g + AHEAD, j)

            rows = rows_bufs[i]
            pos = pos_bufs[i]

            def add_row(jj, c2, rows=rows, pos=pos):
                def add_vec(k, c3):
                    off = k * LANES
                    p = pos[jj, pl.ds(off, LANES)]
                    for b in range(B):
                        plsc.addupdate(rows.at[b * CS + jj, pl.ds(off, LANES)], p)
                    return c3

                return lax.fori_loop(0, VPR, add_vec, c2)

            lax.fori_loop(0, CS, add_row, 0)

            for b in range(B):
                pltpu.async_copy(
                    rows.at[pl.ds(b * CS, CS)],
                    out_hbm.at[b, pl.ds(wseq + g * CS, CS)],
                    ssem[i],
                )

    for i in range(NBUF):
        for b in range(B):
            pltpu.make_async_copy(
                rows_bufs[i].at[pl.ds(b * CS, CS)],
                out_hbm.at[b, pl.ds(0, CS)],
                ssem[i],
            ).wait()


def kernel(x, table, pos_encoding):
    return _embed(x.astype(jnp.int32), table, pos_encoding)
